# baseline (speedup 1.0000x reference)
import functools

import jax
import jax.numpy as jnp
from jax import lax
from jax.experimental import pallas as pl
from jax.experimental.pallas import tpu as pltpu
from jax.experimental.pallas import tpu_sc as plsc

N_NODES = 10000
N_EDGES = 320000
D = 128
DH = D // 2
NUM_CLASSES = 10
NUM_GRAPHS = 128

NC = 2
NS = 16
NW = NC * NS

CHUNK = 128
NBUF = 3

EPS = N_EDGES // NS
NCH = 159
EPS_PAD = NCH * CHUNK
EA_PAD = EPS_PAD * NS

NCHD = 79
ED_PAD = NCHD * CHUNK * NW

NPAD = 10240
SLAB = NPAD // NS

_f32 = jnp.float32
_i32 = jnp.int32

_mesh = plsc.VectorSubcoreMesh(
    core_axis_name="c", subcore_axis_name="s", num_cores=NC, num_subcores=NS)


def _deg_body(dst_hbm, ew_hbm, out_hbm, didx, ewv, zrow, degsh):
    c = lax.axis_index("c")
    s = lax.axis_index("s")
    wid = c * NS + s
    for f in range(SLAB // 16):
        zrow[pl.ds(f * 16, 16)] = jnp.zeros((16,), _f32)
    pltpu.sync_copy(zrow, degsh.at[pl.ds(s * SLAB, SLAB)])
    plsc.subcore_barrier()
    pltpu.sync_copy(dst_hbm.at[wid], didx)
    pltpu.sync_copy(ew_hbm.at[wid], ewv)

    def chunk(j, carry):
        pltpu.sync_copy(ewv.at[j], degsh.at[didx.at[j]], add=True)
        return carry

    lax.fori_loop(0, NCHD, chunk, 0)
    plsc.subcore_barrier()
    pltpu.sync_copy(degsh.at[pl.ds(s * SLAB, SLAB)],
                    out_hbm.at[c, pl.ds(s * SLAB, SLAB)])


_deg_call = pl.kernel(
    _deg_body,
    out_type=jax.ShapeDtypeStruct((NC, NPAD), _f32),
    mesh=_mesh,
    scratch_types=[
        pltpu.VMEM((NCHD, CHUNK), _i32),
        pltpu.VMEM((NCHD, CHUNK), _f32),
        pltpu.VMEM((SLAB,), _f32),
        pltpu.VMEM_SHARED((NPAD,), _f32),
    ],
)


def _agg_body(src_hbm, dst_hbm, ew_hbm, hw2s_hbm, out_hbm,
              sidx, didx, ewv, r0, r1, r2, accsh, s0, s1, s2):
    c = lax.axis_index("c")
    s = lax.axis_index("s")
    rows_l = (r0, r1, r2)
    sems = (s0, s1, s2)

    def zrow_body(i, carry):
        for f in range(DH // 16):
            r0[i, pl.ds(f * 16, 16)] = jnp.zeros((16,), _f32)
        return carry

    lax.fori_loop(0, CHUNK, zrow_body, 0)
    for r in range(SLAB // CHUNK):
        pltpu.sync_copy(r0, accsh.at[pl.ds(s * SLAB + r * CHUNK, CHUNK)])
    plsc.subcore_barrier()

    pltpu.sync_copy(src_hbm.at[s], sidx)
    pltpu.sync_copy(dst_hbm.at[s], didx)
    pltpu.sync_copy(ew_hbm.at[s], ewv)

    half = hw2s_hbm.at[c]

    for b in range(NBUF):
        pltpu.async_copy(half.at[sidx.at[b]], rows_l[b], sems[b])

    def outer(g, carry):
        for b in range(NBUF):
            j = g * NBUF + b
            rows = rows_l[b]
            pltpu.make_async_copy(half.at[sidx.at[j]], rows, sems[b]).wait()

            def grp(t, c2):
                base = t * 16
                wv = ewv[j, pl.ds(base, 16)]
                for kk in range(16):
                    w = wv[kk]
                    for f in range(DH // 16):
                        sl = pl.ds(f * 16, 16)
                        rows[base + kk, sl] = rows[base + kk, sl] * w
                return c2

            lax.fori_loop(0, CHUNK // 16, grp, 0)
            pltpu.sync_copy(rows, accsh.at[didx.at[j]], add=True)
            nxt = j + NBUF

            @pl.when(nxt < NCH)
            def _():
                pltpu.async_copy(half.at[sidx.at[nxt]], rows, sems[b])

        return carry

    lax.fori_loop(0, NCH // NBUF, outer, 0)
    plsc.subcore_barrier()
    pltpu.sync_copy(accsh.at[pl.ds(s * SLAB, SLAB)],
                    out_hbm.at[c, pl.ds(s * SLAB, SLAB)])


_agg_call = pl.kernel(
    _agg_body,
    out_type=jax.ShapeDtypeStruct((NC, NPAD, DH), _f32),
    mesh=_mesh,
    scratch_types=[
        pltpu.VMEM((NCH, CHUNK), _i32),
        pltpu.VMEM((NCH, CHUNK), _i32),
        pltpu.VMEM((NCH, CHUNK), _f32),
        pltpu.VMEM((CHUNK, DH), _f32),
        pltpu.VMEM((CHUNK, DH), _f32),
        pltpu.VMEM((CHUNK, DH), _f32),
        pltpu.VMEM_SHARED((NPAD, DH), _f32),
        pltpu.SemaphoreType.DMA,
        pltpu.SemaphoreType.DMA,
        pltpu.SemaphoreType.DMA,
    ],
    compiler_params=pltpu.CompilerParams(use_tc_tiling_on_sc=False),
)


def _split(hw2, out_ref):
    out_ref[0] = hw2[:, 0:DH]
    out_ref[1] = hw2[:, DH:D]


def _cat(ref):
    return jnp.concatenate([ref[0], ref[1]], axis=1)


def _mmA_body(degT_ref, x_ref, w_ref, dinv_ref, hw2s_ref):
    d = degT_ref[:, 0:1] + degT_ref[:, 1:2] + 1.0
    dinv = lax.rsqrt(d)
    dinv_ref[...] = dinv
    hw = jnp.dot(x_ref[...], w_ref[...], preferred_element_type=_f32)
    _split(dinv[0:N_NODES, :] * hw, hw2s_ref)


_mmA_call = pl.pallas_call(
    _mmA_body,
    out_shape=[
        jax.ShapeDtypeStruct((NPAD, 1), _f32),
        jax.ShapeDtypeStruct((NC, N_NODES, DH), _f32),
    ],
)


def _mmB_body(residual, acc_ref, hw2p_ref, hprev_ref, dinv_ref, b_ref, w_ref,
              h_ref, hw2s_ref):
    agg = jnp.concatenate(
        [acc_ref[0, 0:N_NODES, :], acc_ref[1, 0:N_NODES, :]], axis=1)
    dinv = dinv_ref[0:N_NODES, :]
    pre = dinv * (agg + _cat(hw2p_ref)) + b_ref[...]
    h = jnp.maximum(pre, 0.0)
    if residual:
        h = h + hprev_ref[...]
    h_ref[...] = h
    _split(dinv * jnp.dot(h, w_ref[...], preferred_element_type=_f32),
           hw2s_ref)


def _make_mmB(residual):
    return pl.pallas_call(
        functools.partial(_mmB_body, residual),
        out_shape=[
            jax.ShapeDtypeStruct((N_NODES, D), _f32),
            jax.ShapeDtypeStruct((NC, N_NODES, DH), _f32),
        ],
    )


_mmB1 = _make_mmB(False)
_mmB2 = _make_mmB(True)


def _mmC_body(acc_ref, hw2p_ref, hprev_ref, dinv_ref, b_ref, batch_ref,
              wm_ref, bm_ref, out_ref):
    agg = jnp.concatenate(
        [acc_ref[0, 0:N_NODES, :], acc_ref[1, 0:N_NODES, :]], axis=1)
    dinv = dinv_ref[0:N_NODES, :]
    h = jnp.maximum(dinv * (agg + _cat(hw2p_ref)) + b_ref[...], 0.0)
    h = h + hprev_ref[...]
    bb = jnp.broadcast_to(batch_ref[...], (NUM_GRAPHS, N_NODES))
    gids = lax.broadcasted_iota(_i32, (NUM_GRAPHS, N_NODES), 0)
    pt = (bb == gids).astype(_f32)
    sums = jnp.dot(pt, h, preferred_element_type=_f32)
    cnt = jnp.sum(pt, axis=1, keepdims=True)
    hg = sums / jnp.maximum(cnt, 1.0)
    out_ref[...] = jnp.dot(hg, wm_ref[...], preferred_element_type=_f32) \
        + bm_ref[...]


_mmC_call = pl.pallas_call(
    _mmC_body,
    out_shape=jax.ShapeDtypeStruct((NUM_GRAPHS, NUM_CLASSES), _f32),
)


def kernel(x, edge_index, edge_attr, edge_weight, batch,
           W0, b0, W1, b1, W2, b2, Wm, bm):
    src = edge_index[0].astype(_i32)
    dst = edge_index[1].astype(_i32)
    ew = edge_weight.astype(_f32)

    pad_a = EA_PAD - N_EDGES
    srcp = jnp.concatenate([src, jnp.zeros((pad_a,), _i32)]).reshape(NS, NCH, CHUNK)
    dstp = jnp.concatenate([dst, jnp.zeros((pad_a,), _i32)]).reshape(NS, NCH, CHUNK)
    ewp = jnp.concatenate([ew, jnp.zeros((pad_a,), _f32)]).reshape(NS, NCH, CHUNK)

    pad_d = ED_PAD - N_EDGES
    dstd = jnp.concatenate([dst, jnp.zeros((pad_d,), _i32)]).reshape(NW, NCHD, CHUNK)
    ewd = jnp.concatenate([ew, jnp.zeros((pad_d,), _f32)]).reshape(NW, NCHD, CHUNK)

    degpair = _deg_call(dstd, ewd)
    dinv, hw2_0 = _mmA_call(degpair.T, x, W0)
    acc0 = _agg_call(srcp, dstp, ewp, hw2_0)
    h1, hw2_1 = _mmB1(acc0, hw2_0, x, dinv, b0.reshape(1, D), W1)
    acc1 = _agg_call(srcp, dstp, ewp, hw2_1)
    h2, hw2_2 = _mmB2(acc1, hw2_1, h1, dinv, b1.reshape(1, D), W2)
    acc2 = _agg_call(srcp, dstp, ewp, hw2_2)
    out = _mmC_call(acc2, hw2_2, h2, dinv, b2.reshape(1, D),
                    batch.astype(_i32).reshape(1, N_NODES),
                    Wm, bm.reshape(1, NUM_CLASSES))
    return out


# speedup vs baseline: 1.1297x; 1.1297x over previous
import functools

import jax
import jax.numpy as jnp
from jax import lax
from jax.experimental import pallas as pl
from jax.experimental.pallas import tpu as pltpu
from jax.experimental.pallas import tpu_sc as plsc

N_NODES = 10000
N_EDGES = 320000
D = 128
DH = D // 2
NUM_CLASSES = 10
NUM_GRAPHS = 128

NC = 2
NS = 16
NW = NC * NS

CHUNK = 128
NBUF = 4
EB = 2 * NBUF

EPS = N_EDGES // NS
NCH = -(-EPS // CHUNK)
EPS_PAD = NCH * CHUNK
EA_PAD = EPS_PAD * NS

NCHD = 79
ED_PAD = NCHD * CHUNK * NW

NPAD = 10240
SLAB = NPAD // NS

_f32 = jnp.float32
_i32 = jnp.int32

_mesh = plsc.VectorSubcoreMesh(
    core_axis_name="c", subcore_axis_name="s", num_cores=NC, num_subcores=NS)


def _deg_body(dst_hbm, ew_hbm, out_hbm, didx, ewv, zrow, degsh):
    c = lax.axis_index("c")
    s = lax.axis_index("s")
    wid = c * NS + s
    for f in range(SLAB // 16):
        zrow[pl.ds(f * 16, 16)] = jnp.zeros((16,), _f32)
    pltpu.sync_copy(zrow, degsh.at[pl.ds(s * SLAB, SLAB)])
    plsc.subcore_barrier()
    pltpu.sync_copy(dst_hbm.at[wid], didx)
    pltpu.sync_copy(ew_hbm.at[wid], ewv)

    def chunk(j, carry):
        pltpu.sync_copy(ewv.at[j], degsh.at[didx.at[j]], add=True)
        return carry

    lax.fori_loop(0, NCHD, chunk, 0)
    plsc.subcore_barrier()
    pltpu.sync_copy(degsh.at[pl.ds(s * SLAB, SLAB)],
                    out_hbm.at[c, pl.ds(s * SLAB, SLAB)])


_deg_call = pl.kernel(
    _deg_body,
    out_type=jax.ShapeDtypeStruct((NC, NPAD), _f32),
    mesh=_mesh,
    scratch_types=[
        pltpu.VMEM((NCHD, CHUNK), _i32),
        pltpu.VMEM((NCHD, CHUNK), _f32),
        pltpu.VMEM((SLAB,), _f32),
        pltpu.VMEM_SHARED((NPAD,), _f32),
    ],
)


def _agg_body(edata_hbm, ew_hbm, hw2s_hbm, out_hbm, ga, sb, edb, ewb, accsh,
              gsem, ssem, esem):
    c = lax.axis_index("c")
    s = lax.axis_index("s")

    def zrow_body(i, carry):
        for f in range(DH // 16):
            ga[0, i, pl.ds(f * 16, 16)] = jnp.zeros((16,), _f32)
        return carry

    lax.fori_loop(0, CHUNK, zrow_body, 0)
    for r in range(SLAB // CHUNK):
        pltpu.sync_copy(ga.at[0], accsh.at[pl.ds(s * SLAB + r * CHUNK, CHUNK)])
    plsc.subcore_barrier()

    half = hw2s_hbm.at[c]
    eslice = edata_hbm.at[s]
    wslice = ew_hbm.at[s]

    def load_edata(j, slot):
        pltpu.async_copy(eslice.at[j], edb.at[slot], esem.at[slot])
        pltpu.async_copy(wslice.at[j], ewb.at[slot], esem.at[slot])

    def wait_edata(slot):
        pltpu.make_async_copy(eslice.at[0], edb.at[slot], esem.at[slot]).wait()
        pltpu.make_async_copy(wslice.at[0], ewb.at[slot], esem.at[slot]).wait()

    for j in range(EB):
        load_edata(j, j)
    for j in range(NBUF):
        wait_edata(j)
        pltpu.async_copy(half.at[edb.at[j, 0]], ga.at[j], gsem.at[j])

    def step(j, carry):
        b = lax.rem(j, NBUF)
        e = lax.rem(j, EB)
        pltpu.make_async_copy(half.at[edb.at[e, 0]], ga.at[b],
                              gsem.at[b]).wait()

        @pl.when(j >= NBUF)
        def _():
            pltpu.make_async_copy(sb.at[b], accsh.at[edb.at[e, 1]],
                                  ssem.at[b]).wait()
            nxt = j + NBUF

            @pl.when(nxt < NCH)
            def _():
                load_edata(nxt, lax.rem(nxt, EB))

        def grp(t, c2):
            base = t * 16
            wv = ewb[e, pl.ds(base, 16)]
            for kk in range(16):
                w = wv[kk]
                for f in range(DH // 16):
                    sl = pl.ds(f * 16, 16)
                    sb[b, base + kk, sl] = ga[b, base + kk, sl] * w
            return c2

        lax.fori_loop(0, CHUNK // 16, grp, 0)

        pltpu.async_copy(sb.at[b], accsh.at[edb.at[e, 1]], ssem.at[b],
                         add=True)

        nx = j + NBUF

        @pl.when(nx < NCH)
        def _():
            en = lax.rem(nx, EB)
            wait_edata(en)
            pltpu.async_copy(half.at[edb.at[en, 0]], ga.at[b], gsem.at[b])

        return carry

    lax.fori_loop(0, NCH, step, 0)
    for b in range(NBUF):
        pltpu.make_async_copy(sb.at[b], accsh.at[edb.at[0, 1]],
                              ssem.at[b]).wait()
    plsc.subcore_barrier()
    pltpu.sync_copy(accsh.at[pl.ds(s * SLAB, SLAB)],
                    out_hbm.at[c, pl.ds(s * SLAB, SLAB)])


_agg_call = pl.kernel(
    _agg_body,
    out_type=jax.ShapeDtypeStruct((NC, NPAD, DH), _f32),
    mesh=_mesh,
    scratch_types=[
        pltpu.VMEM((NBUF, CHUNK, DH), _f32),
        pltpu.VMEM((NBUF, CHUNK, DH), _f32),
        pltpu.VMEM((EB, 2, CHUNK), _i32),
        pltpu.VMEM((EB, CHUNK), _f32),
        pltpu.VMEM_SHARED((NPAD, DH), _f32),
        pltpu.SemaphoreType.DMA((NBUF,)),
        pltpu.SemaphoreType.DMA((NBUF,)),
        pltpu.SemaphoreType.DMA((EB,)),
    ],
    compiler_params=pltpu.CompilerParams(use_tc_tiling_on_sc=False),
)


def _split(hw2, out_ref):
    out_ref[0] = hw2[:, 0:DH]
    out_ref[1] = hw2[:, DH:D]


def _cat(ref):
    return jnp.concatenate([ref[0], ref[1]], axis=1)


def _mmA_body(degT_ref, x_ref, w_ref, dinv_ref, hw2s_ref):
    d = degT_ref[:, 0:1] + degT_ref[:, 1:2] + 1.0
    dinv = lax.rsqrt(d)
    dinv_ref[...] = dinv
    hw = jnp.dot(x_ref[...], w_ref[...], preferred_element_type=_f32)
    _split(dinv[0:N_NODES, :] * hw, hw2s_ref)


_mmA_call = pl.pallas_call(
    _mmA_body,
    out_shape=[
        jax.ShapeDtypeStruct((NPAD, 1), _f32),
        jax.ShapeDtypeStruct((NC, N_NODES, DH), _f32),
    ],
)


def _mmB_body(residual, acc_ref, hw2p_ref, hprev_ref, dinv_ref, b_ref, w_ref,
              h_ref, hw2s_ref):
    agg = jnp.concatenate(
        [acc_ref[0, 0:N_NODES, :], acc_ref[1, 0:N_NODES, :]], axis=1)
    dinv = dinv_ref[0:N_NODES, :]
    pre = dinv * (agg + _cat(hw2p_ref)) + b_ref[...]
    h = jnp.maximum(pre, 0.0)
    if residual:
        h = h + hprev_ref[...]
    h_ref[...] = h
    _split(dinv * jnp.dot(h, w_ref[...], preferred_element_type=_f32),
           hw2s_ref)


def _make_mmB(residual):
    return pl.pallas_call(
        functools.partial(_mmB_body, residual),
        out_shape=[
            jax.ShapeDtypeStruct((N_NODES, D), _f32),
            jax.ShapeDtypeStruct((NC, N_NODES, DH), _f32),
        ],
    )


_mmB1 = _make_mmB(False)
_mmB2 = _make_mmB(True)


def _mmC_body(acc_ref, hw2p_ref, hprev_ref, dinv_ref, b_ref, batch_ref,
              wm_ref, bm_ref, out_ref):
    agg = jnp.concatenate(
        [acc_ref[0, 0:N_NODES, :], acc_ref[1, 0:N_NODES, :]], axis=1)
    dinv = dinv_ref[0:N_NODES, :]
    h = jnp.maximum(dinv * (agg + _cat(hw2p_ref)) + b_ref[...], 0.0)
    h = h + hprev_ref[...]
    bb = jnp.broadcast_to(batch_ref[...], (NUM_GRAPHS, N_NODES))
    gids = lax.broadcasted_iota(_i32, (NUM_GRAPHS, N_NODES), 0)
    pt = (bb == gids).astype(_f32)
    sums = jnp.dot(pt, h, preferred_element_type=_f32)
    cnt = jnp.sum(pt, axis=1, keepdims=True)
    hg = sums / jnp.maximum(cnt, 1.0)
    out_ref[...] = jnp.dot(hg, wm_ref[...], preferred_element_type=_f32) \
        + bm_ref[...]


_mmC_call = pl.pallas_call(
    _mmC_body,
    out_shape=jax.ShapeDtypeStruct((NUM_GRAPHS, NUM_CLASSES), _f32),
)


def kernel(x, edge_index, edge_attr, edge_weight, batch,
           W0, b0, W1, b1, W2, b2, Wm, bm):
    src = edge_index[0].astype(_i32)
    dst = edge_index[1].astype(_i32)
    ew = edge_weight.astype(_f32)

    pad_a = EA_PAD - N_EDGES
    srcp = jnp.concatenate([src, jnp.zeros((pad_a,), _i32)]).reshape(NS, NCH, CHUNK)
    dstp = jnp.concatenate([dst, jnp.zeros((pad_a,), _i32)]).reshape(NS, NCH, CHUNK)
    ewp = jnp.concatenate([ew, jnp.zeros((pad_a,), _f32)]).reshape(NS, NCH, CHUNK)
    edata = jnp.stack([srcp, dstp], axis=2)

    pad_d = ED_PAD - N_EDGES
    dstd = jnp.concatenate([dst, jnp.zeros((pad_d,), _i32)]).reshape(NW, NCHD, CHUNK)
    ewd = jnp.concatenate([ew, jnp.zeros((pad_d,), _f32)]).reshape(NW, NCHD, CHUNK)

    degpair = _deg_call(dstd, ewd)
    dinv, hw2_0 = _mmA_call(degpair.T, x, W0)
    acc0 = _agg_call(edata, ewp, hw2_0)
    h1, hw2_1 = _mmB1(acc0, hw2_0, x, dinv, b0.reshape(1, D), W1)
    acc1 = _agg_call(edata, ewp, hw2_1)
    h2, hw2_2 = _mmB2(acc1, hw2_1, h1, dinv, b1.reshape(1, D), W2)
    acc2 = _agg_call(edata, ewp, hw2_2)
    out = _mmC_call(acc2, hw2_2, h2, dinv, b2.reshape(1, D),
                    batch.astype(_i32).reshape(1, N_NODES),
                    Wm, bm.reshape(1, NUM_CLASSES))
    return out


# speedup vs baseline: 1.4324x; 1.2680x over previous
import functools

import jax
import jax.numpy as jnp
from jax import lax
from jax.experimental import pallas as pl
from jax.experimental.pallas import tpu as pltpu
from jax.experimental.pallas import tpu_sc as plsc

N_NODES = 10000
N_EDGES = 320000
D = 128
DH = D // 2
NUM_CLASSES = 10
NUM_GRAPHS = 128

NC = 2
NS = 16
NW = NC * NS

CHUNK = 128
NBUF = 4
EB = 2 * NBUF

EPS = N_EDGES // NS
NCH = 160
EPS_PAD = NCH * CHUNK
EA_PAD = EPS_PAD * NS

NCHD = 79
ED_PAD = NCHD * CHUNK * NW

NPAD = 10240
SLAB = NPAD // NS

_f32 = jnp.float32
_i32 = jnp.int32

_mesh = plsc.VectorSubcoreMesh(
    core_axis_name="c", subcore_axis_name="s", num_cores=NC, num_subcores=NS)


def _deg_body(dst_hbm, ew_hbm, out_hbm, didx, ewv, zrow, degsh):
    c = lax.axis_index("c")
    s = lax.axis_index("s")
    wid = c * NS + s
    for f in range(SLAB // 16):
        zrow[pl.ds(f * 16, 16)] = jnp.zeros((16,), _f32)
    pltpu.sync_copy(zrow, degsh.at[pl.ds(s * SLAB, SLAB)])
    plsc.subcore_barrier()
    pltpu.sync_copy(dst_hbm.at[wid], didx)
    pltpu.sync_copy(ew_hbm.at[wid], ewv)

    def chunk(j, carry):
        pltpu.sync_copy(ewv.at[j], degsh.at[didx.at[j]], add=True)
        return carry

    lax.fori_loop(0, NCHD, chunk, 0)
    plsc.subcore_barrier()
    pltpu.sync_copy(degsh.at[pl.ds(s * SLAB, SLAB)],
                    out_hbm.at[c, pl.ds(s * SLAB, SLAB)])


_deg_call = pl.kernel(
    _deg_body,
    out_type=jax.ShapeDtypeStruct((NC, NPAD), _f32),
    mesh=_mesh,
    scratch_types=[
        pltpu.VMEM((NCHD, CHUNK), _i32),
        pltpu.VMEM((NCHD, CHUNK), _f32),
        pltpu.VMEM((SLAB,), _f32),
        pltpu.VMEM_SHARED((NPAD,), _f32),
    ],
)


def _agg_body(edata_hbm, ew_hbm, hw2s_hbm, out_hbm, ga, sb, edb, ewb, accsh,
              gsem, ssem, esem):
    c = lax.axis_index("c")
    s = lax.axis_index("s")

    def zrow_body(i, carry):
        for f in range(DH // 16):
            ga[0, i, pl.ds(f * 16, 16)] = jnp.zeros((16,), _f32)
        return carry

    lax.fori_loop(0, CHUNK, zrow_body, 0)
    for r in range(SLAB // CHUNK):
        pltpu.sync_copy(ga.at[0], accsh.at[pl.ds(s * SLAB + r * CHUNK, CHUNK)])
    plsc.subcore_barrier()

    half = hw2s_hbm.at[c]
    eslice = edata_hbm.at[s]
    wslice = ew_hbm.at[s]

    def load_edata(j, slot):
        pltpu.async_copy(eslice.at[j], edb.at[slot], esem.at[slot])
        pltpu.async_copy(wslice.at[j], ewb.at[slot], esem.at[slot])

    def wait_edata(slot):
        pltpu.make_async_copy(eslice.at[0], edb.at[slot], esem.at[slot]).wait()
        pltpu.make_async_copy(wslice.at[0], ewb.at[slot], esem.at[slot]).wait()

    for j in range(EB):
        load_edata(j, j)
    for j in range(NBUF):
        wait_edata(j)
        pltpu.async_copy(half.at[edb.at[j, 0]], ga.at[j], gsem.at[j])

    def step(g, carry):
        for b in range(NBUF):
            j = g * NBUF + b
            e = lax.rem(j, EB)
            pltpu.make_async_copy(half.at[edb.at[e, 0]], ga.at[b],
                                  gsem.at[b]).wait()

            @pl.when(j >= NBUF)
            def _():
                pltpu.make_async_copy(sb.at[b], accsh.at[edb.at[e, 1]],
                                      ssem.at[b]).wait()
                nxt = j + NBUF

                @pl.when(nxt < NCH)
                def _():
                    load_edata(nxt, lax.rem(nxt, EB))

            def grp(t, c2):
                base = t * 16
                wv = ewb[e, pl.ds(base, 16)]
                for kk in range(16):
                    w = wv[kk]
                    for f in range(DH // 16):
                        sl = pl.ds(f * 16, 16)
                        sb[b, base + kk, sl] = ga[b, base + kk, sl] * w
                return c2

            lax.fori_loop(0, CHUNK // 16, grp, 0)

            pltpu.async_copy(sb.at[b], accsh.at[edb.at[e, 1]], ssem.at[b],
                             add=True)

            nx = j + NBUF

            @pl.when(nx < NCH)
            def _():
                en = lax.rem(nx, EB)
                wait_edata(en)
                pltpu.async_copy(half.at[edb.at[en, 0]], ga.at[b], gsem.at[b])

        return carry

    lax.fori_loop(0, NCH // NBUF, step, 0)
    for b in range(NBUF):
        pltpu.make_async_copy(sb.at[b], accsh.at[edb.at[0, 1]],
                              ssem.at[b]).wait()
    plsc.subcore_barrier()
    pltpu.sync_copy(accsh.at[pl.ds(s * SLAB, SLAB)],
                    out_hbm.at[c, pl.ds(s * SLAB, SLAB)])


_agg_call = pl.kernel(
    _agg_body,
    out_type=jax.ShapeDtypeStruct((NC, NPAD, DH), _f32),
    mesh=_mesh,
    scratch_types=[
        pltpu.VMEM((NBUF, CHUNK, DH), _f32),
        pltpu.VMEM((NBUF, CHUNK, DH), _f32),
        pltpu.VMEM((EB, 2, CHUNK), _i32),
        pltpu.VMEM((EB, CHUNK), _f32),
        pltpu.VMEM_SHARED((NPAD, DH), _f32),
        pltpu.SemaphoreType.DMA((NBUF,)),
        pltpu.SemaphoreType.DMA((NBUF,)),
        pltpu.SemaphoreType.DMA((EB,)),
    ],
    compiler_params=pltpu.CompilerParams(use_tc_tiling_on_sc=False),
)


def _split(hw2, out_ref):
    out_ref[0] = hw2[:, 0:DH]
    out_ref[1] = hw2[:, DH:D]


def _cat(ref):
    return jnp.concatenate([ref[0], ref[1]], axis=1)


def _mmA_body(degT_ref, x_ref, w_ref, dinv_ref, hw2s_ref):
    d = degT_ref[:, 0:1] + degT_ref[:, 1:2] + 1.0
    dinv = lax.rsqrt(d)
    dinv_ref[...] = dinv
    hw = jnp.dot(x_ref[...], w_ref[...], preferred_element_type=_f32)
    _split(dinv[0:N_NODES, :] * hw, hw2s_ref)


_mmA_call = pl.pallas_call(
    _mmA_body,
    out_shape=[
        jax.ShapeDtypeStruct((NPAD, 1), _f32),
        jax.ShapeDtypeStruct((NC, N_NODES, DH), _f32),
    ],
)


def _mmB_body(residual, acc_ref, hw2p_ref, hprev_ref, dinv_ref, b_ref, w_ref,
              h_ref, hw2s_ref):
    agg = jnp.concatenate(
        [acc_ref[0, 0:N_NODES, :], acc_ref[1, 0:N_NODES, :]], axis=1)
    dinv = dinv_ref[0:N_NODES, :]
    pre = dinv * (agg + _cat(hw2p_ref)) + b_ref[...]
    h = jnp.maximum(pre, 0.0)
    if residual:
        h = h + hprev_ref[...]
    h_ref[...] = h
    _split(dinv * jnp.dot(h, w_ref[...], preferred_element_type=_f32),
           hw2s_ref)


def _make_mmB(residual):
    return pl.pallas_call(
        functools.partial(_mmB_body, residual),
        out_shape=[
            jax.ShapeDtypeStruct((N_NODES, D), _f32),
            jax.ShapeDtypeStruct((NC, N_NODES, DH), _f32),
        ],
    )


_mmB1 = _make_mmB(False)
_mmB2 = _make_mmB(True)


def _mmC_body(acc_ref, hw2p_ref, hprev_ref, dinv_ref, b_ref, batch_ref,
              wm_ref, bm_ref, out_ref):
    agg = jnp.concatenate(
        [acc_ref[0, 0:N_NODES, :], acc_ref[1, 0:N_NODES, :]], axis=1)
    dinv = dinv_ref[0:N_NODES, :]
    h = jnp.maximum(dinv * (agg + _cat(hw2p_ref)) + b_ref[...], 0.0)
    h = h + hprev_ref[...]
    bb = jnp.broadcast_to(batch_ref[...], (NUM_GRAPHS, N_NODES))
    gids = lax.broadcasted_iota(_i32, (NUM_GRAPHS, N_NODES), 0)
    pt = (bb == gids).astype(_f32)
    sums = jnp.dot(pt, h, preferred_element_type=_f32)
    cnt = jnp.sum(pt, axis=1, keepdims=True)
    hg = sums / jnp.maximum(cnt, 1.0)
    out_ref[...] = jnp.dot(hg, wm_ref[...], preferred_element_type=_f32) \
        + bm_ref[...]


_mmC_call = pl.pallas_call(
    _mmC_body,
    out_shape=jax.ShapeDtypeStruct((NUM_GRAPHS, NUM_CLASSES), _f32),
)


def kernel(x, edge_index, edge_attr, edge_weight, batch,
           W0, b0, W1, b1, W2, b2, Wm, bm):
    src = edge_index[0].astype(_i32)
    dst = edge_index[1].astype(_i32)
    ew = edge_weight.astype(_f32)

    pad_a = EA_PAD - N_EDGES
    srcp = jnp.concatenate([src, jnp.zeros((pad_a,), _i32)]).reshape(NS, NCH, CHUNK)
    dstp = jnp.concatenate([dst, jnp.zeros((pad_a,), _i32)]).reshape(NS, NCH, CHUNK)
    ewp = jnp.concatenate([ew, jnp.zeros((pad_a,), _f32)]).reshape(NS, NCH, CHUNK)
    edata = jnp.stack([srcp, dstp], axis=2)

    pad_d = ED_PAD - N_EDGES
    dstd = jnp.concatenate([dst, jnp.zeros((pad_d,), _i32)]).reshape(NW, NCHD, CHUNK)
    ewd = jnp.concatenate([ew, jnp.zeros((pad_d,), _f32)]).reshape(NW, NCHD, CHUNK)

    degpair = _deg_call(dstd, ewd)
    dinv, hw2_0 = _mmA_call(degpair.T, x, W0)
    acc0 = _agg_call(edata, ewp, hw2_0)
    h1, hw2_1 = _mmB1(acc0, hw2_0, x, dinv, b0.reshape(1, D), W1)
    acc1 = _agg_call(edata, ewp, hw2_1)
    h2, hw2_2 = _mmB2(acc1, hw2_1, h1, dinv, b1.reshape(1, D), W2)
    acc2 = _agg_call(edata, ewp, hw2_2)
    out = _mmC_call(acc2, hw2_2, h2, dinv, b2.reshape(1, D),
                    batch.astype(_i32).reshape(1, N_NODES),
                    Wm, bm.reshape(1, NUM_CLASSES))
    return out
